# baseline (speedup 1.0000x reference)
import functools

import jax
import jax.numpy as jnp
from jax import lax
from jax.experimental import pallas as pl
from jax.experimental.pallas import tpu as pltpu
from jax.experimental.pallas import tpu_sc as plsc

_VOCAB = 1000000
_EMBED = 64
_B = 1024
_L = 512
_NW = 32
_SEQ_PER_W = _B // _NW
_CHUNK = 128
_NCH = _L // _CHUNK
_NTF = _VOCAB // 128
_TAIL = _VOCAB - _NTF * 128
_TPW = (_NTF + _NW - 1) // _NW
_PROWS = _VOCAB // 2

_mesh = plsc.VectorSubcoreMesh(core_axis_name="c", subcore_axis_name="s")


def _wid():
    return lax.axis_index("s") * 2 + lax.axis_index("c")


@functools.partial(
    pl.kernel,
    out_type=jax.ShapeDtypeStruct((_PROWS, 128), jnp.float32),
    mesh=_mesh,
    scratch_types=[
        pltpu.VMEM((2, 64, 128), jnp.float32),
        pltpu.VMEM((2, 64, 128), jnp.float32),
        pltpu.SemaphoreType.DMA,
        pltpu.SemaphoreType.DMA,
        pltpu.SemaphoreType.DMA,
        pltpu.SemaphoreType.DMA,
    ],
    compiler_params=pltpu.CompilerParams(
        use_tc_tiling_on_sc=True, needs_layout_passes=False,
        disable_bounds_checks=True,
    ),
)
def _format_table(wt_hbm, tail_hbm, out_hbm, a_v, b_v, rsem0, rsem1,
                  wsem0, wsem1):
    w = _wid()
    j16 = lax.iota(jnp.int32, 16)
    rsems = (rsem0, rsem1)
    wsems = (wsem0, wsem1)
    yqs, ycs = [], []
    for d in range(16):
        y = (j16 + d) & 15
        yqs.append(y >> 1)
        ycs.append((y & 1) * 64 + j16)

    def tile_t(i):
        return w + _NW * i

    def read_cp(i, slot):
        return pltpu.make_async_copy(
            wt_hbm.at[:, pl.ds(tile_t(i) * 128, 128)], a_v.at[slot],
            rsems[slot],
        )

    def write_cp(i, slot):
        return pltpu.make_async_copy(
            b_v.at[slot], out_hbm.at[pl.ds(tile_t(i) * 64, 64)], wsems[slot]
        )

    def transpose_block(slot, nv16):
        a = a_v.at[slot]
        b = b_v.at[slot]

        @plsc.parallel_loop(0, nv16, unroll=4)
        def vloop(vi):
            q0 = vi * 8
            for e0 in (0, 16, 32, 48):
                idx_e = j16 + e0
                for d in range(16):
                    idx_v = vi * 16 + ((j16 + d) & 15)
                    vec = plsc.load_gather(a, [idx_e, idx_v])
                    plsc.store_scatter(b, [yqs[d] + q0, ycs[d] + e0], vec)

    read_cp(0, 0).start()

    def body(g, carry):
        for s in (0, 1):
            i = 2 * g + s

            @pl.when(tile_t(i + 1) < _NTF)
            def _(i=i, s=s):
                read_cp(i + 1, 1 - s).start()

            @pl.when(tile_t(i) < _NTF)
            def _(i=i, s=s):
                read_cp(i, s).wait()

                @pl.when(i >= 2)
                def _():
                    write_cp(i - 2, s).wait()

                transpose_block(s, 8)
                write_cp(i, s).start()

        return carry

    lax.fori_loop(0, (_TPW + 1) // 2, body, 0)
    for back in (2, 1):
        i = _TPW - back

        @pl.when(tile_t(i) < _NTF)
        def _(i=i):
            write_cp(i, i % 2).wait()

    @pl.when(w == 0)
    def _():
        pltpu.sync_copy(tail_hbm, a_v.at[0])
        transpose_block(0, _TAIL // 16)
        pltpu.sync_copy(
            b_v.at[0, pl.ds(0, _TAIL // 2)],
            out_hbm.at[pl.ds(_NTF * 64, _TAIL // 2)],
        )


@functools.partial(
    pl.kernel,
    out_type=jax.ShapeDtypeStruct((_B, 8, _NCH, 1024), jnp.float32),
    mesh=_mesh,
    scratch_types=[
        pltpu.VMEM((_NCH, _CHUNK), jnp.int32),
        pltpu.VMEM((2 * _NCH, _CHUNK), jnp.int32),
        pltpu.VMEM((2, 2 * _CHUNK, 32), jnp.float32),
        pltpu.VMEM((2, 8192), jnp.float32),
        pltpu.VMEM((_NCH, 8192), jnp.float32),
        pltpu.SemaphoreType.DMA,
        pltpu.SemaphoreType.DMA,
        pltpu.SemaphoreType.DMA,
        pltpu.SemaphoreType.DMA,
    ],
    compiler_params=pltpu.CompilerParams(
        use_tc_tiling_on_sc=False, needs_layout_passes=False,
        disable_bounds_checks=True,
    ),
)
def _emb_lookup(ids_hbm, tbl_hbm, pos_hbm, out_hbm, idx_v, pidx_v,
                rows_v, blk_v, pos_v, gsem0, gsem1, wsem0, wsem1):
    w = _wid()
    j16 = lax.iota(jnp.int32, 16)
    gsems = (gsem0, gsem1)
    wsems = (wsem0, wsem1)
    pltpu.sync_copy(pos_hbm, pos_v)
    dpat, ypat = [], []
    for d in range(16):
        y = (j16 + d) & 15
        dpat.append((y >> 3) * 1024 + (y & 7) * 128 + j16)
        ypat.append(y)

    def stage_seq(b):
        pltpu.sync_copy(ids_hbm.at[b], idx_v)
        for j in range(_NCH):
            for k in range(_CHUNK // 16):
                sl = pl.ds(k * 16, 16)
                v2 = idx_v[j, sl] * 2
                row = 2 * j + (0 if k < 4 else 1)
                rvec = jnp.full((16,), row, jnp.int32)
                base = 32 * k if k < 4 else 32 * k - 128
                plsc.store_scatter(pidx_v, [rvec, base + 2 * j16], v2)
                plsc.store_scatter(pidx_v, [rvec, base + 2 * j16 + 1], v2 + 1)

    def gather_cps(c, slot):
        return [
            pltpu.make_async_copy(
                tbl_hbm.at[pidx_v.at[2 * c + h]],
                rows_v.at[slot, pl.ds(h * _CHUNK, _CHUNK)],
                gsems[slot],
            )
            for h in (0, 1)
        ]

    def write_cps(b, c, slot):
        return [
            pltpu.make_async_copy(
                blk_v.at[slot, pl.ds(et * 1024, 1024)],
                out_hbm.at[b, et, c],
                wsems[slot],
            )
            for et in range(8)
        ]

    def transpose_chunk(c, slot):
        rows = rows_v.at[slot]
        blk = blk_v.at[slot]

        @plsc.parallel_loop(0, 512, unroll=8)
        def initloop(k):
            sl = pl.ds(k * 16, 16)
            blk[sl] = pos_v[c, sl]

        @plsc.parallel_loop(0, 8, unroll=2)
        def lloop(li):
            l0 = li * 16
            idx_l2 = (j16 + l0) * 2
            for d in range(16):
                yd = ypat[d]
                for e0 in (0, 16, 32, 48):
                    idx_r = idx_l2 + (1 if e0 >= 32 else 0)
                    vec = plsc.load_gather(rows, [idx_r, yd + (e0 & 31)])
                    plsc.addupdate_scatter(blk, [dpat[d] + (e0 * 128 + l0)], vec)

    def seq_body(si, carry):
        b = w * _SEQ_PER_W + si
        stage_seq(b)
        for cp in gather_cps(0, 0):
            cp.start()
        for c in range(_NCH):
            s = c % 2
            if c + 1 < _NCH:
                for cp in gather_cps(c + 1, (c + 1) % 2):
                    cp.start()
            for cp in gather_cps(c, s):
                cp.wait()

            @pl.when(si > 0)
            def _(c=c, s=s):
                for cp in write_cps(b - 1, c + 2 if c < 2 else c, s):
                    cp.wait()

            @pl.when((si == 0) & (c >= 2))
            def _(c=c, s=s):
                for cp in write_cps(b, c - 2, s):
                    cp.wait()

            transpose_chunk(c, s)
            for cp in write_cps(b, c, s):
                cp.start()
        return carry

    lax.fori_loop(0, _SEQ_PER_W, seq_body, 0)
    b_last = w * _SEQ_PER_W + _SEQ_PER_W - 1
    for c in (2, 3):
        for cp in write_cps(b_last, c, c % 2):
            cp.wait()


def kernel(input_ids, word_table, pos_table):
    ids3d = input_ids.reshape(_B, _NCH, _CHUNK)
    wt = word_table.T
    tail = jnp.pad(wt[:, _NTF * 128:], ((0, 0), (0, 128 - _TAIL)))
    table_pairs = _format_table(wt, tail).reshape(2 * _VOCAB, 32)
    pos4 = (
        pos_table.reshape(_NCH, _CHUNK, 8, 8)
        .transpose(0, 2, 3, 1)
        .reshape(_NCH, 8192)
    )
    out = _emb_lookup(ids3d, table_pairs, pos4)
    return (
        out.reshape(_B, 8, _NCH, 8, _CHUNK)
        .transpose(0, 2, 4, 1, 3)
        .reshape(_B, _L, _EMBED)
    )


# speedup vs baseline: 1.5370x; 1.5370x over previous
import functools

import jax
import jax.numpy as jnp
from jax import lax
from jax.experimental import pallas as pl
from jax.experimental.pallas import tpu as pltpu
from jax.experimental.pallas import tpu_sc as plsc

_VOCAB = 1000000
_EMBED = 64
_B = 1024
_L = 512
_NW = 32
_SEQ_PER_W = _B // _NW
_CHUNK = 128
_NCH = _L // _CHUNK
_NTF = _VOCAB // 128
_TAIL = _VOCAB - _NTF * 128
_TPW = (_NTF + _NW - 1) // _NW
_PROWS = _VOCAB // 2

_mesh = plsc.VectorSubcoreMesh(core_axis_name="c", subcore_axis_name="s")


def _wid():
    return lax.axis_index("s") * 2 + lax.axis_index("c")


@functools.partial(
    pl.kernel,
    out_type=jax.ShapeDtypeStruct((_PROWS, 128), jnp.float32),
    mesh=_mesh,
    scratch_types=[
        pltpu.VMEM((2, 64, 128), jnp.float32),
        pltpu.VMEM((2, 64, 128), jnp.float32),
        pltpu.SemaphoreType.DMA,
        pltpu.SemaphoreType.DMA,
        pltpu.SemaphoreType.DMA,
        pltpu.SemaphoreType.DMA,
    ],
    compiler_params=pltpu.CompilerParams(
        use_tc_tiling_on_sc=True, needs_layout_passes=False,
        disable_bounds_checks=True,
    ),
)
def _format_table(wt_hbm, tail_hbm, out_hbm, a_v, b_v, rsem0, rsem1,
                  wsem0, wsem1):
    w = _wid()
    j16 = lax.iota(jnp.int32, 16)
    rsems = (rsem0, rsem1)
    wsems = (wsem0, wsem1)
    yqs, ycs = [], []
    for d in range(16):
        y = (j16 + d) & 15
        yqs.append(y >> 1)
        ycs.append((y & 1) * 64 + j16)

    def tile_t(i):
        return w + _NW * i

    def read_cp(i, slot):
        return pltpu.make_async_copy(
            wt_hbm.at[:, pl.ds(tile_t(i) * 128, 128)], a_v.at[slot],
            rsems[slot],
        )

    def write_cp(i, slot):
        return pltpu.make_async_copy(
            b_v.at[slot], out_hbm.at[pl.ds(tile_t(i) * 64, 64)], wsems[slot]
        )

    def transpose_block(slot, nv16):
        a = a_v.at[slot]
        b = b_v.at[slot]

        @plsc.parallel_loop(0, nv16, unroll=4)
        def vloop(vi):
            q0 = vi * 8
            for e0 in (0, 16, 32, 48):
                idx_e = j16 + e0
                for d in range(16):
                    idx_v = vi * 16 + ((j16 + d) & 15)
                    vec = plsc.load_gather(a, [idx_e, idx_v])
                    plsc.store_scatter(b, [yqs[d] + q0, ycs[d] + e0], vec)

    read_cp(0, 0).start()

    def body(g, carry):
        for s in (0, 1):
            i = 2 * g + s

            @pl.when(tile_t(i + 1) < _NTF)
            def _(i=i, s=s):
                read_cp(i + 1, 1 - s).start()

            @pl.when(tile_t(i) < _NTF)
            def _(i=i, s=s):
                read_cp(i, s).wait()

                @pl.when(i >= 2)
                def _():
                    write_cp(i - 2, s).wait()

                transpose_block(s, 8)
                write_cp(i, s).start()

        return carry

    lax.fori_loop(0, (_TPW + 1) // 2, body, 0)
    for back in (2, 1):
        i = _TPW - back

        @pl.when(tile_t(i) < _NTF)
        def _(i=i):
            write_cp(i, i % 2).wait()

    @pl.when(w == 0)
    def _():
        pltpu.sync_copy(tail_hbm, a_v.at[0])
        transpose_block(0, _TAIL // 16)
        pltpu.sync_copy(
            b_v.at[0, pl.ds(0, _TAIL // 2)],
            out_hbm.at[pl.ds(_NTF * 64, _TAIL // 2)],
        )


@functools.partial(
    pl.kernel,
    out_type=jax.ShapeDtypeStruct((_B, 8, _NCH, 1024), jnp.float32),
    mesh=_mesh,
    scratch_types=[
        pltpu.VMEM((_NCH, _CHUNK), jnp.int32),
        pltpu.VMEM((_NCH, _CHUNK), jnp.int32),
        pltpu.VMEM((_NCH, _CHUNK), jnp.int32),
        pltpu.VMEM((2, _CHUNK, 128), jnp.float32),
        pltpu.VMEM((2, 8192), jnp.float32),
        pltpu.VMEM((_NCH, 8192), jnp.float32),
        pltpu.SemaphoreType.DMA,
        pltpu.SemaphoreType.DMA,
        pltpu.SemaphoreType.DMA,
        pltpu.SemaphoreType.DMA,
    ],
    compiler_params=pltpu.CompilerParams(
        use_tc_tiling_on_sc=False, needs_layout_passes=False,
        disable_bounds_checks=True,
    ),
)
def _emb_lookup(ids_hbm, tbl_hbm, pos_hbm, out_hbm, idx_v, pidx_v, h64_v,
                rows_v, blk_v, pos_v, gsem0, gsem1, wsem0, wsem1):
    w = _wid()
    j16 = lax.iota(jnp.int32, 16)
    gsems = (gsem0, gsem1)
    wsems = (wsem0, wsem1)
    pltpu.sync_copy(pos_hbm, pos_v)
    dpat, ypat = [], []
    for d in range(16):
        y = (j16 + d) & 15
        dpat.append((y >> 3) * 1024 + (y & 7) * 128 + j16)
        ypat.append(y)

    def stage_seq(b):
        pltpu.sync_copy(ids_hbm.at[b], idx_v)
        for j in range(_NCH):
            for k in range(_CHUNK // 16):
                sl = pl.ds(k * 16, 16)
                v = idx_v[j, sl]
                pidx_v[j, sl] = v >> 1
                h64_v[j, sl] = (v & 1) * 64

    def gather_cps(c, slot):
        return [
            pltpu.make_async_copy(
                tbl_hbm.at[pidx_v.at[c]], rows_v.at[slot], gsems[slot]
            )
        ]

    def write_cps(b, c, slot):
        return [
            pltpu.make_async_copy(
                blk_v.at[slot, pl.ds(et * 1024, 1024)],
                out_hbm.at[b, et, c],
                wsems[slot],
            )
            for et in range(8)
        ]

    def transpose_chunk(c, slot):
        rows = rows_v.at[slot]
        blk = blk_v.at[slot]

        @plsc.parallel_loop(0, 512, unroll=8)
        def initloop(k):
            sl = pl.ds(k * 16, 16)
            blk[sl] = pos_v[c, sl]

        @plsc.parallel_loop(0, 8, unroll=2)
        def lloop(li):
            l0 = li * 16
            idx_l = j16 + l0
            h64s = h64_v[c, pl.ds(l0, 16)]
            for d in range(16):
                hd = h64s + ypat[d]
                for e0 in (0, 16, 32, 48):
                    vec = plsc.load_gather(rows, [idx_l, hd + e0])
                    plsc.addupdate_scatter(blk, [dpat[d] + (e0 * 128 + l0)], vec)

    def seq_body(si, carry):
        b = w * _SEQ_PER_W + si
        stage_seq(b)
        for cp in gather_cps(0, 0):
            cp.start()
        for c in range(_NCH):
            s = c % 2
            if c + 1 < _NCH:
                for cp in gather_cps(c + 1, (c + 1) % 2):
                    cp.start()
            for cp in gather_cps(c, s):
                cp.wait()

            @pl.when(si > 0)
            def _(c=c, s=s):
                for cp in write_cps(b - 1, c + 2 if c < 2 else c, s):
                    cp.wait()

            @pl.when((si == 0) & (c >= 2))
            def _(c=c, s=s):
                for cp in write_cps(b, c - 2, s):
                    cp.wait()

            transpose_chunk(c, s)
            for cp in write_cps(b, c, s):
                cp.start()
        return carry

    lax.fori_loop(0, _SEQ_PER_W, seq_body, 0)
    b_last = w * _SEQ_PER_W + _SEQ_PER_W - 1
    for c in (2, 3):
        for cp in write_cps(b_last, c, c % 2):
            cp.wait()


def kernel(input_ids, word_table, pos_table):
    ids3d = input_ids.reshape(_B, _NCH, _CHUNK)
    wt = word_table.T
    tail = jnp.pad(wt[:, _NTF * 128:], ((0, 0), (0, 128 - _TAIL)))
    table_pairs = _format_table(wt, tail)
    pos4 = (
        pos_table.reshape(_NCH, _CHUNK, 8, 8)
        .transpose(0, 2, 3, 1)
        .reshape(_NCH, 8192)
    )
    out = _emb_lookup(ids3d, table_pairs, pos4)
    return (
        out.reshape(_B, 8, _NCH, 8, _CHUNK)
        .transpose(0, 2, 4, 1, 3)
        .reshape(_B, _L, _EMBED)
    )
